# baseline (speedup 1.0000x reference)
import jax
import jax.numpy as jnp
from jax.experimental import pallas as pl
from jax.experimental.pallas import tpu as pltpu


def _hard_attention_kernel(feat_ref, hid_ref, wf_ref, bf_ref, wh_ref, bh_ref,
                           wst_ref, alpha_ref, z_ref, lo_scr, best_scr, m_scr,
                           *, bn, nb_total):
    nb = pl.program_id(1)
    feat = feat_ref[0]
    hrow = hid_ref[0]
    hb = jnp.dot(hrow, wh_ref[...], preferred_element_type=jnp.float32) + bh_ref[...]
    fw = jnp.dot(feat, wf_ref[...], preferred_element_type=jnp.float32) + bf_ref[...]
    u = jnp.tanh(fw + hb)
    lo = jax.lax.dot_general(wst_ref[...], u, (((1,), (1,)), ((), ())),
                             preferred_element_type=jnp.float32)
    lo_scr[:, pl.ds(nb * bn, bn)] = lo

    bm = jnp.max(lo)
    iota = jax.lax.broadcasted_iota(jnp.int32, (1, bn), 1)
    bidx = jnp.min(jnp.where(lo == bm, iota, bn))
    onehot = (iota == bidx).astype(jnp.float32)
    row = jnp.dot(onehot, feat, preferred_element_type=jnp.float32,
                  precision=jax.lax.Precision.HIGHEST)

    @pl.when(nb == 0)
    def _init():
        m_scr[0, 0] = bm
        best_scr[...] = row

    @pl.when(nb > 0)
    def _update():
        prev = m_scr[0, 0]
        better = bm > prev
        m_scr[0, 0] = jnp.where(better, bm, prev)
        best_scr[...] = jnp.where(better, row, best_scr[...])

    @pl.when(nb == nb_total - 1)
    def _finalize():
        lo_all = lo_scr[...]
        mx = jnp.max(lo_all)
        e = jnp.exp(lo_all - mx)
        s = jnp.sum(e)
        alpha_ref[0] = e * (1.0 / s)
        z_ref[0] = best_scr[...]


def kernel(features, hidden, Wf, bf, Wh, bh, Ws, bs):
    B, N, D = features.shape
    H = hidden.shape[1]
    U = Wf.shape[1]
    BN = 512
    NB = N // BN

    hidden3 = hidden.reshape(B, 1, H)
    bf2 = bf.reshape(1, U)
    bh2 = bh.reshape(1, U)
    wst = Ws.reshape(1, U)

    import functools
    body = functools.partial(_hard_attention_kernel, bn=BN, nb_total=NB)

    alpha_out, z_out = pl.pallas_call(
        body,
        grid=(B, NB),
        in_specs=[
            pl.BlockSpec((1, BN, D), lambda b, nb: (b, nb, 0)),
            pl.BlockSpec((1, 1, H), lambda b, nb: (b, 0, 0)),
            pl.BlockSpec((D, U), lambda b, nb: (0, 0)),
            pl.BlockSpec((1, U), lambda b, nb: (0, 0)),
            pl.BlockSpec((H, U), lambda b, nb: (0, 0)),
            pl.BlockSpec((1, U), lambda b, nb: (0, 0)),
            pl.BlockSpec((1, U), lambda b, nb: (0, 0)),
        ],
        out_specs=[
            pl.BlockSpec((1, 1, N), lambda b, nb: (b, 0, 0)),
            pl.BlockSpec((1, 1, D), lambda b, nb: (b, 0, 0)),
        ],
        out_shape=[
            jax.ShapeDtypeStruct((B, 1, N), jnp.float32),
            jax.ShapeDtypeStruct((B, 1, D), jnp.float32),
        ],
        scratch_shapes=[
            pltpu.VMEM((1, N), jnp.float32),
            pltpu.VMEM((1, D), jnp.float32),
            pltpu.SMEM((1, 1), jnp.float32),
        ],
    )(features, hidden3, Wf, bf2, Wh, bh2, wst)

    alpha = alpha_out.reshape(B, N, 1)
    z = z_out.reshape(B, D)
    return z, alpha


# speedup vs baseline: 3.4475x; 3.4475x over previous
import functools

import jax
import jax.numpy as jnp
from jax.experimental import pallas as pl
from jax.experimental.pallas import tpu as pltpu


def _logits_kernel(feat_ref, hid_ref, wft_ref, bf_ref, wh_ref, bh_ref,
                   wst_ref, lo_ref):
    feat = feat_ref[0]
    hrow = hid_ref[0]
    ft = jax.lax.dot_general(wft_ref[...], feat, (((1,), (1,)), ((), ())),
                             preferred_element_type=jnp.float32)
    hb_row = (jnp.dot(hrow, wh_ref[...], preferred_element_type=jnp.float32)
              + bh_ref[...] + bf_ref[...])
    ones_row = jnp.ones((1, ft.shape[1]), jnp.float32)
    hb_bc = jax.lax.dot_general(hb_row, ones_row, (((0,), (0,)), ((), ())),
                                preferred_element_type=jnp.float32,
                                precision=jax.lax.Precision.HIGHEST)
    u = jnp.tanh(ft + hb_bc)
    lo_ref[0, 0] = jnp.dot(wst_ref[...], u, preferred_element_type=jnp.float32)


def _softmax_argmax_kernel(lo_ref, alpha_ref, loc_ref, *, n):
    lo = lo_ref[...]
    m = jnp.max(lo, axis=1, keepdims=True)
    e = jnp.exp(lo - m)
    s = jnp.sum(e, axis=1, keepdims=True)
    alpha_ref[...] = e * (1.0 / s)
    iota = jax.lax.broadcasted_iota(jnp.int32, lo.shape, 1)
    loc_ref[...] = jnp.min(jnp.where(lo == m, iota, n), axis=1, keepdims=True)


def _gather_kernel(loc_ref, feat_ref, z_ref):
    z_ref[...] = feat_ref[0]


def kernel(features, hidden, Wf, bf, Wh, bh, Ws, bs):
    B, N, D = features.shape
    H = hidden.shape[1]
    U = Wf.shape[1]
    BN = 2048
    NB = N // BN
    BB = 8

    hidden3 = hidden.reshape(B, 1, H)
    wft = Wf.T
    bf2 = bf.reshape(1, U)
    bh2 = bh.reshape(1, U)
    wst = Ws.reshape(1, U)

    lo_out = pl.pallas_call(
        _logits_kernel,
        grid=(B, NB),
        in_specs=[
            pl.BlockSpec((1, BN, D), lambda b, nb: (b, nb, 0)),
            pl.BlockSpec((1, 1, H), lambda b, nb: (b, 0, 0)),
            pl.BlockSpec((U, D), lambda b, nb: (0, 0)),
            pl.BlockSpec((1, U), lambda b, nb: (0, 0)),
            pl.BlockSpec((H, U), lambda b, nb: (0, 0)),
            pl.BlockSpec((1, U), lambda b, nb: (0, 0)),
            pl.BlockSpec((1, U), lambda b, nb: (0, 0)),
        ],
        out_specs=pl.BlockSpec((1, 1, 1, BN), lambda b, nb: (b, nb, 0, 0)),
        out_shape=jax.ShapeDtypeStruct((B, NB, 1, BN), jnp.float32),
        compiler_params=pltpu.CompilerParams(
            dimension_semantics=("parallel", "parallel")),
    )(features, hidden3, wft, bf2, Wh, bh2, wst)

    logits = lo_out.reshape(B, N)

    alpha2, loc2 = pl.pallas_call(
        functools.partial(_softmax_argmax_kernel, n=N),
        grid=(B // BB,),
        in_specs=[pl.BlockSpec((BB, N), lambda i: (i, 0))],
        out_specs=[
            pl.BlockSpec((BB, N), lambda i: (i, 0)),
            pl.BlockSpec((BB, 1), lambda i: (i, 0)),
        ],
        out_shape=[
            jax.ShapeDtypeStruct((B, N), jnp.float32),
            jax.ShapeDtypeStruct((B, 1), jnp.int32),
        ],
        compiler_params=pltpu.CompilerParams(
            dimension_semantics=("parallel",)),
    )(logits)

    loc = loc2.reshape(B)
    feat4 = features.reshape(B, N, 1, D)

    grid_spec = pltpu.PrefetchScalarGridSpec(
        num_scalar_prefetch=1,
        grid=(B,),
        in_specs=[
            pl.BlockSpec((1, 1, 1, D), lambda b, loc_ref: (b, loc_ref[b], 0, 0)),
        ],
        out_specs=pl.BlockSpec((1, 1, D), lambda b, loc_ref: (b, 0, 0)),
    )
    z3 = pl.pallas_call(
        _gather_kernel,
        grid_spec=grid_spec,
        out_shape=jax.ShapeDtypeStruct((B, 1, D), jnp.float32),
    )(loc, feat4)

    alpha = alpha2.reshape(B, N, 1)
    z = z3.reshape(B, D)
    return z, alpha


# speedup vs baseline: 4.3323x; 1.2566x over previous
import jax
import jax.numpy as jnp
from jax.experimental import pallas as pl
from jax.experimental.pallas import tpu as pltpu


def _hard_attention_kernel(feat_ref, hid_ref, wft_ref, bf_ref, wh_ref, bh_ref,
                           wst_ref, alpha_ref, z_ref):
    feat = feat_ref[0]
    hrow = hid_ref[0]
    n = feat.shape[0]

    ft = jax.lax.dot_general(wft_ref[...], feat, (((1,), (1,)), ((), ())),
                             preferred_element_type=jnp.float32)
    hb_row = (jnp.dot(hrow, wh_ref[...], preferred_element_type=jnp.float32)
              + bh_ref[...] + bf_ref[...])
    ones_row = jnp.ones((1, n), jnp.float32)
    hb_bc = jax.lax.dot_general(hb_row, ones_row, (((0,), (0,)), ((), ())),
                                preferred_element_type=jnp.float32,
                                precision=jax.lax.Precision.HIGHEST)
    u = jnp.tanh(ft + hb_bc)
    lo = jnp.dot(wst_ref[...], u, preferred_element_type=jnp.float32)

    m = jnp.max(lo)
    e = jnp.exp(lo - m)
    s = jnp.sum(e)
    alpha_ref[0] = e * (1.0 / s)

    iota = jax.lax.broadcasted_iota(jnp.int32, (1, n), 1)
    bidx = jnp.min(jnp.where(lo == m, iota, n))
    onehot = (iota == bidx).astype(jnp.float32)
    z_ref[0] = jnp.dot(onehot, feat, preferred_element_type=jnp.float32,
                       precision=jax.lax.Precision.HIGHEST)


def kernel(features, hidden, Wf, bf, Wh, bh, Ws, bs):
    B, N, D = features.shape
    H = hidden.shape[1]
    U = Wf.shape[1]

    hidden3 = hidden.reshape(B, 1, H)
    wft = Wf.T
    bf2 = bf.reshape(1, U)
    bh2 = bh.reshape(1, U)
    wst = Ws.reshape(1, U)

    alpha2, z3 = pl.pallas_call(
        _hard_attention_kernel,
        grid=(B,),
        in_specs=[
            pl.BlockSpec((1, N, D), lambda b: (b, 0, 0)),
            pl.BlockSpec((1, 1, H), lambda b: (b, 0, 0)),
            pl.BlockSpec((U, D), lambda b: (0, 0)),
            pl.BlockSpec((1, U), lambda b: (0, 0)),
            pl.BlockSpec((H, U), lambda b: (0, 0)),
            pl.BlockSpec((1, U), lambda b: (0, 0)),
            pl.BlockSpec((1, U), lambda b: (0, 0)),
        ],
        out_specs=[
            pl.BlockSpec((1, 1, N), lambda b: (b, 0, 0)),
            pl.BlockSpec((1, 1, D), lambda b: (b, 0, 0)),
        ],
        out_shape=[
            jax.ShapeDtypeStruct((B, 1, N), jnp.float32),
            jax.ShapeDtypeStruct((B, 1, D), jnp.float32),
        ],
        compiler_params=pltpu.CompilerParams(
            dimension_semantics=("parallel",)),
    )(features, hidden3, wft, bf2, Wh, bh2, wst)

    alpha = alpha2.reshape(B, N, 1)
    z = z3.reshape(B, D)
    return z, alpha


# speedup vs baseline: 8.2103x; 1.8951x over previous
import jax
import jax.numpy as jnp
from jax.experimental import pallas as pl
from jax.experimental.pallas import tpu as pltpu


def _hard_attention_kernel(feat_ref, hid_ref, wft_ref, bf_ref, wh_ref, bh_ref,
                           wst_ref, alpha_ref, z_ref):
    feat = feat_ref[0]
    hrow = hid_ref[0]
    n = feat.shape[0]

    ft = jax.lax.dot_general(wft_ref[...], feat, (((1,), (1,)), ((), ())),
                             preferred_element_type=jnp.float32)
    hb_row = (jnp.dot(hrow, wh_ref[...], preferred_element_type=jnp.float32)
              + bh_ref[...] + bf_ref[...])
    ones_row = jnp.ones((1, n), jnp.float32)
    hb_bc = jax.lax.dot_general(hb_row, ones_row, (((0,), (0,)), ((), ())),
                                preferred_element_type=jnp.float32,
                                precision=jax.lax.Precision.HIGHEST)
    u = jnp.tanh(ft + hb_bc)
    lo = jnp.dot(wst_ref[...], u, preferred_element_type=jnp.float32)

    m = jnp.max(lo)
    e = jnp.exp(lo - m)
    s = jnp.sum(e)
    alpha_ref[0] = e * (1.0 / s)

    iota = jax.lax.broadcasted_iota(jnp.int32, (1, n), 1)
    bidx = jnp.min(jnp.where(lo == m, iota, n))
    z_ref[0] = feat_ref[0, pl.ds(bidx, 1), :]


def kernel(features, hidden, Wf, bf, Wh, bh, Ws, bs):
    B, N, D = features.shape
    H = hidden.shape[1]
    U = Wf.shape[1]

    hidden3 = hidden.reshape(B, 1, H)
    wft = Wf.T
    bf2 = bf.reshape(1, U)
    bh2 = bh.reshape(1, U)
    wst = Ws.reshape(1, U)

    alpha2, z3 = pl.pallas_call(
        _hard_attention_kernel,
        grid=(B,),
        in_specs=[
            pl.BlockSpec((1, N, D), lambda b: (b, 0, 0)),
            pl.BlockSpec((1, 1, H), lambda b: (b, 0, 0)),
            pl.BlockSpec((U, D), lambda b: (0, 0)),
            pl.BlockSpec((1, U), lambda b: (0, 0)),
            pl.BlockSpec((H, U), lambda b: (0, 0)),
            pl.BlockSpec((1, U), lambda b: (0, 0)),
            pl.BlockSpec((1, U), lambda b: (0, 0)),
        ],
        out_specs=[
            pl.BlockSpec((1, 1, N), lambda b: (b, 0, 0)),
            pl.BlockSpec((1, 1, D), lambda b: (b, 0, 0)),
        ],
        out_shape=[
            jax.ShapeDtypeStruct((B, 1, N), jnp.float32),
            jax.ShapeDtypeStruct((B, 1, D), jnp.float32),
        ],
        compiler_params=pltpu.CompilerParams(
            dimension_semantics=("parallel",)),
    )(features, hidden3, wft, bf2, Wh, bh2, wst)

    alpha = alpha2.reshape(B, N, 1)
    z = z3.reshape(B, D)
    return z, alpha


# speedup vs baseline: 8.2659x; 1.0068x over previous
import jax
import jax.numpy as jnp
from jax.experimental import pallas as pl
from jax.experimental.pallas import tpu as pltpu


def _hard_attention_kernel(feat_ref, hid_ref, wft_ref, bf_ref, wh_ref, bh_ref,
                           wst_ref, alpha_ref, z_ref):
    b = pl.program_id(0)
    feat = feat_ref[0]
    hrow = hid_ref[pl.ds(b, 1), 0, :]
    n = feat.shape[0]

    ft = jax.lax.dot_general(wft_ref[...], feat, (((1,), (1,)), ((), ())),
                             preferred_element_type=jnp.float32)
    hb_row = (jnp.dot(hrow, wh_ref[...], preferred_element_type=jnp.float32)
              + bh_ref[...] + bf_ref[...])
    ones_row = jnp.ones((1, n), jnp.float32)
    hb_bc = jax.lax.dot_general(hb_row, ones_row, (((0,), (0,)), ((), ())),
                                preferred_element_type=jnp.float32,
                                precision=jax.lax.Precision.HIGHEST)
    u = jnp.tanh(ft + hb_bc)
    lo = jnp.dot(wst_ref[...], u, preferred_element_type=jnp.float32)

    m = jnp.max(lo)
    e = jnp.exp(lo - m)
    s = jnp.sum(e)
    alpha_ref[pl.ds(b, 1), 0, :] = e * (1.0 / s)

    iota = jax.lax.broadcasted_iota(jnp.int32, (1, n), 1)
    bidx = jnp.min(jnp.where(lo == m, iota, n))
    z_ref[pl.ds(b, 1), 0, :] = feat_ref[0, pl.ds(bidx, 1), :]


def kernel(features, hidden, Wf, bf, Wh, bh, Ws, bs):
    B, N, D = features.shape
    H = hidden.shape[1]
    U = Wf.shape[1]

    hidden3 = hidden.reshape(B, 1, H)
    wft = Wf.T
    bf2 = bf.reshape(1, U)
    bh2 = bh.reshape(1, U)
    wst = Ws.reshape(1, U)

    alpha2, z3 = pl.pallas_call(
        _hard_attention_kernel,
        grid=(B,),
        in_specs=[
            pl.BlockSpec((1, N, D), lambda b: (b, 0, 0)),
            pl.BlockSpec((B, 1, H), lambda b: (0, 0, 0)),
            pl.BlockSpec((U, D), lambda b: (0, 0)),
            pl.BlockSpec((1, U), lambda b: (0, 0)),
            pl.BlockSpec((H, U), lambda b: (0, 0)),
            pl.BlockSpec((1, U), lambda b: (0, 0)),
            pl.BlockSpec((1, U), lambda b: (0, 0)),
        ],
        out_specs=[
            pl.BlockSpec((B, 1, N), lambda b: (0, 0, 0)),
            pl.BlockSpec((B, 1, D), lambda b: (0, 0, 0)),
        ],
        out_shape=[
            jax.ShapeDtypeStruct((B, 1, N), jnp.float32),
            jax.ShapeDtypeStruct((B, 1, D), jnp.float32),
        ],
        compiler_params=pltpu.CompilerParams(
            dimension_semantics=("arbitrary",)),
    )(features, hidden3, wft, bf2, Wh, bh2, wst)

    alpha = alpha2.reshape(B, N, 1)
    z = z3.reshape(B, D)
    return z, alpha
